# baseline (speedup 1.0000x reference)
import functools

import jax
import jax.numpy as jnp
from jax import lax
from jax.experimental import pallas as pl
from jax.experimental.pallas import tpu as pltpu
from jax.experimental.pallas import tpu_sc as plsc

VOCAB = 49408
N_WORDS = 77
D = 768
B = 1024

NW = 32
CHUNK = B // NW
LANES = 16
D_SLICES = D // LANES


def _sc_embed(tok_ids, pos_ids, tok_emb, pos_emb):
    mesh = plsc.VectorSubcoreMesh(core_axis_name="c", subcore_axis_name="s")

    @functools.partial(
        pl.kernel,
        mesh=mesh,
        out_type=jax.ShapeDtypeStruct((N_WORDS, B, D), jnp.float32),
        scratch_types=[
            pltpu.VMEM((N_WORDS, CHUNK), jnp.int32),
            pltpu.VMEM((N_WORDS, CHUNK), jnp.int32),
            pltpu.VMEM((2, CHUNK, D), jnp.float32),
            pltpu.VMEM((N_WORDS, D), jnp.float32),
            pltpu.SemaphoreType.DMA,
            pltpu.SemaphoreType.DMA,
            pltpu.SemaphoreType.DMA,
            pltpu.SemaphoreType.DMA,
        ],
    )
    def k(tok_ids_hbm, pos_ids_hbm, tok_emb_hbm, pos_emb_hbm, out_hbm,
          tok_idx_v, pos_idx_v, trows, ptab_v,
          gsem0, gsem1, wsem0, wsem1):
        wid = lax.axis_index("s") * 2 + lax.axis_index("c")
        col0 = wid * CHUNK
        pltpu.sync_copy(tok_ids_hbm.at[wid], tok_idx_v)
        pltpu.sync_copy(pos_ids_hbm.at[wid], pos_idx_v)
        pltpu.sync_copy(pos_emb_hbm, ptab_v)

        pltpu.async_copy(tok_emb_hbm.at[tok_idx_v.at[0]], trows.at[0], gsem0)

        def make_pre(cur, gsem_c, nxt, gsem_n, wsem_n):
            def pre(i):
                @pl.when(i + 1 < N_WORDS)
                def _issue_next():
                    @pl.when(i >= 1)
                    def _drain_write():
                        pltpu.make_async_copy(
                            trows.at[nxt], out_hbm.at[0, pl.ds(col0, CHUNK)],
                            wsem_n).wait()
                    pltpu.async_copy(
                        tok_emb_hbm.at[tok_idx_v.at[i + 1]],
                        trows.at[nxt], gsem_n)

                pltpu.make_async_copy(
                    tok_emb_hbm.at[tok_idx_v.at[i]],
                    trows.at[cur], gsem_c).wait()
            return pre

        def make_post(cur, wsem_c):
            def post(i):
                pltpu.async_copy(
                    trows.at[cur], out_hbm.at[i, pl.ds(col0, CHUNK)], wsem_c)
            return post

        pre_even = make_pre(0, gsem0, 1, gsem1, wsem1)
        pre_odd = make_pre(1, gsem1, 0, gsem0, wsem0)
        post_even = make_post(0, wsem0)
        post_odd = make_post(1, wsem1)

        def word_body(i, carry):
            par = i % 2

            @pl.when(par == 0)
            def _even_pre():
                pre_even(i)

            @pl.when(par == 1)
            def _odd_pre():
                pre_odd(i)

            def group_body(g, pid_vec):
                for l in range(LANES):
                    pid = pid_vec[l]
                    r = g * LANES + l
                    for j in range(D_SLICES):
                        sl = pl.ds(j * LANES, LANES)
                        trows[par, r, sl] = trows[par, r, sl] + ptab_v[pid, sl]

            group_body(0, pos_idx_v[i, pl.ds(0, LANES)])
            group_body(1, pos_idx_v[i, pl.ds(LANES, LANES)])

            @pl.when(par == 0)
            def _even_post():
                post_even(i)

            @pl.when(par == 1)
            def _odd_post():
                post_odd(i)
            return carry

        lax.fori_loop(0, N_WORDS, word_body, 0)

        pltpu.make_async_copy(
            trows.at[0], out_hbm.at[0, pl.ds(col0, CHUNK)], wsem0).wait()
        pltpu.make_async_copy(
            trows.at[1], out_hbm.at[0, pl.ds(col0, CHUNK)], wsem1).wait()

    return k(tok_ids, pos_ids, tok_emb, pos_emb)


def kernel(input_ids, position_ids, token_embedding, position_embedding):
    tok_ids = (input_ids.astype(jnp.int32)
               .reshape(NW, CHUNK, N_WORDS)
               .transpose(0, 2, 1))
    pos_ids = (position_ids.astype(jnp.int32)
               .reshape(NW, CHUNK, N_WORDS)
               .transpose(0, 2, 1))
    out = _sc_embed(tok_ids, pos_ids, token_embedding, position_embedding)
    return out.transpose(1, 0, 2)


# speedup vs baseline: 2.2682x; 2.2682x over previous
import functools

import jax
import jax.numpy as jnp
from jax import lax
from jax.experimental import pallas as pl
from jax.experimental.pallas import tpu as pltpu
from jax.experimental.pallas import tpu_sc as plsc

VOCAB = 49408
N_WORDS = 77
D = 768
B = 1024

NW = 32
CHUNK = B // NW
LANES = 16
D_SLICES = D // LANES


def _sc_embed(tok_ids, pos_ids, tok_emb, pos_emb):
    mesh = plsc.VectorSubcoreMesh(core_axis_name="c", subcore_axis_name="s")

    @functools.partial(
        pl.kernel,
        mesh=mesh,
        out_type=jax.ShapeDtypeStruct((N_WORDS, B, D), jnp.float32),
        scratch_types=[
            pltpu.VMEM((N_WORDS, CHUNK), jnp.int32),
            pltpu.VMEM((N_WORDS, CHUNK), jnp.int32),
            pltpu.VMEM((CHUNK, D), jnp.float32),
            pltpu.VMEM((CHUNK, D), jnp.float32),
            pltpu.VMEM((CHUNK, D), jnp.float32),
            pltpu.VMEM((CHUNK, D), jnp.float32),
            pltpu.SemaphoreType.DMA,
            pltpu.SemaphoreType.DMA,
            pltpu.SemaphoreType.DMA,
            pltpu.SemaphoreType.DMA,
        ],
    )
    def k(tok_ids_hbm, pos_ids_hbm, tok_emb_hbm, pos_emb_hbm, out_hbm,
          tok_idx_v, pos_idx_v, trows0, trows1, prows0, prows1,
          gsem0, gsem1, wsem0, wsem1):
        wid = lax.axis_index("s") * 2 + lax.axis_index("c")
        col0 = wid * CHUNK
        pltpu.sync_copy(tok_ids_hbm.at[wid], tok_idx_v)
        pltpu.sync_copy(pos_ids_hbm.at[wid], pos_idx_v)

        pltpu.async_copy(tok_emb_hbm.at[tok_idx_v.at[0]], trows0, gsem0)
        pltpu.async_copy(pos_emb_hbm.at[pos_idx_v.at[0]], prows0, gsem0)

        def make_step(tcur, pcur, gsem_c, wsem_c, tnxt, pnxt, gsem_n, wsem_n):
            def step(i):
                @pl.when(i + 1 < N_WORDS)
                def _issue_next():
                    @pl.when(i >= 1)
                    def _drain_write():
                        pltpu.make_async_copy(
                            tnxt, out_hbm.at[0, pl.ds(col0, CHUNK)],
                            wsem_n).wait()
                    pltpu.async_copy(
                        tok_emb_hbm.at[tok_idx_v.at[i + 1]], tnxt, gsem_n)
                    pltpu.async_copy(
                        pos_emb_hbm.at[pos_idx_v.at[i + 1]], pnxt, gsem_n)

                pltpu.make_async_copy(
                    tok_emb_hbm.at[tok_idx_v.at[i]], tcur, gsem_c).wait()
                pltpu.make_async_copy(
                    pos_emb_hbm.at[pos_idx_v.at[i]], pcur, gsem_c).wait()

                def row_body(r, carry):
                    for j in range(D_SLICES):
                        sl = pl.ds(j * LANES, LANES)
                        tcur[r, sl] = tcur[r, sl] + pcur[r, sl]
                    return carry

                half = CHUNK // 2
                lax.fori_loop(0, half, row_body, 0)
                pltpu.async_copy(
                    tcur.at[pl.ds(0, half)],
                    out_hbm.at[i, pl.ds(col0, half)], wsem_c)
                lax.fori_loop(half, CHUNK, row_body, 0)
                pltpu.async_copy(
                    tcur.at[pl.ds(half, half)],
                    out_hbm.at[i, pl.ds(col0 + half, half)], wsem_c)
            return step

        step_even = make_step(trows0, prows0, gsem0, wsem0,
                              trows1, prows1, gsem1, wsem1)
        step_odd = make_step(trows1, prows1, gsem1, wsem1,
                             trows0, prows0, gsem0, wsem0)

        def word_body(i, carry):
            @pl.when(i % 2 == 0)
            def _even():
                step_even(i)

            @pl.when(i % 2 == 1)
            def _odd():
                step_odd(i)
            return carry

        lax.fori_loop(0, N_WORDS, word_body, 0)

        pltpu.make_async_copy(
            trows0, out_hbm.at[0, pl.ds(col0, CHUNK)], wsem0).wait()
        pltpu.make_async_copy(
            trows1, out_hbm.at[0, pl.ds(col0, CHUNK)], wsem1).wait()

    return k(tok_ids, pos_ids, tok_emb, pos_emb)


def kernel(input_ids, position_ids, token_embedding, position_embedding):
    tok_ids = (input_ids.astype(jnp.int32)
               .reshape(NW, CHUNK, N_WORDS)
               .transpose(0, 2, 1)
               .reshape(NW, N_WORDS, CHUNK))
    pos_ids = (position_ids.astype(jnp.int32)
               .reshape(NW, CHUNK, N_WORDS)
               .transpose(0, 2, 1)
               .reshape(NW, N_WORDS, CHUNK))
    out = _sc_embed(tok_ids, pos_ids, token_embedding, position_embedding)
    return out.transpose(1, 0, 2)
